# Initial kernel scaffold; baseline (speedup 1.0000x reference)
import jax
import jax.numpy as jnp
from jax.experimental import pallas as pl
from jax.experimental.pallas import tpu as pltpu
from jax.experimental.pallas import tpu_sc as plsc

_N_E = 8192
_E_DIM = 32
_N_TOK = 32768

_TILE = 512
_CHUNK = 2048


def _argmin_body(z_ref, wt_ref, zsq_ref, wsq_ref, idx_ref, pxp_ref, counts_ref):
    step = pl.program_id(0)
    nsteps = pl.num_programs(0)

    @pl.when(step == 0)
    def _():
        counts_ref[...] = jnp.zeros_like(counts_ref)

    z = z_ref[...]
    zsq = zsq_ref[...]

    best_val = jnp.full((_TILE, 1), jnp.inf, dtype=jnp.float32)
    best_idx = jnp.zeros((_TILE, 1), dtype=jnp.int32)
    for c in range(_N_E // _CHUNK):
        wt_c = wt_ref[:, c * _CHUNK:(c + 1) * _CHUNK]
        wsq_c = wsq_ref[:, c * _CHUNK:(c + 1) * _CHUNK]
        mm = jnp.dot(z, wt_c, preferred_element_type=jnp.float32)
        d = zsq + wsq_c - 2.0 * mm
        m = jnp.min(d, axis=1, keepdims=True)
        iota = jax.lax.broadcasted_iota(jnp.int32, d.shape, 1) + c * _CHUNK
        i = jnp.min(jnp.where(d == m, iota, jnp.int32(_N_E)), axis=1,
                    keepdims=True)
        upd = m < best_val
        best_val = jnp.where(upd, m, best_val)
        best_idx = jnp.where(upd, i, best_idx)

    idx_ref[...] = best_idx

    for c in range(_N_E // _CHUNK):
        iota = jax.lax.broadcasted_iota(jnp.int32, (_TILE, _CHUNK), 1) + c * _CHUNK
        hit = (best_idx == iota).astype(jnp.float32)
        counts_ref[:, c * _CHUNK:(c + 1) * _CHUNK] += jnp.sum(
            hit, axis=0, keepdims=True)

    @pl.when(step == nsteps - 1)
    def _():
        e_mean = counts_ref[...] * (1.0 / _N_TOK)
        ent = e_mean * jnp.log(e_mean + 1e-10)
        pxp_ref[0, 0] = jnp.exp(-jnp.sum(ent))


def _tc_argmin(z, wt, zsq, wsq):
    n_tiles = _N_TOK // _TILE
    return pl.pallas_call(
        _argmin_body,
        grid=(n_tiles,),
        in_specs=[
            pl.BlockSpec((_TILE, _E_DIM), lambda i: (i, 0)),
            pl.BlockSpec((_E_DIM, _N_E), lambda i: (0, 0)),
            pl.BlockSpec((_TILE, 1), lambda i: (i, 0)),
            pl.BlockSpec((1, _N_E), lambda i: (0, 0)),
        ],
        out_specs=[
            pl.BlockSpec((_TILE, 1), lambda i: (i, 0)),
            pl.BlockSpec((1, 1), lambda i: (0, 0)),
        ],
        out_shape=[
            jax.ShapeDtypeStruct((_N_TOK, 1), jnp.int32),
            jax.ShapeDtypeStruct((1, 1), jnp.float32),
        ],
        scratch_shapes=[pltpu.VMEM((1, _N_E), jnp.float32)],
    )(z, wt, zsq, wsq)


def _sc_gather(W, idx2d):
    win = 128
    mesh = plsc.VectorSubcoreMesh(core_axis_name="c", subcore_axis_name="s")

    @pl.kernel(
        out_type=jax.ShapeDtypeStruct((_N_TOK, _E_DIM), W.dtype),
        mesh=mesh,
    )
    def k(w_hbm, i_hbm, o_hbm):
        def body(i_vmem, o_vmem):
            pltpu.sync_copy(w_hbm.at[i_vmem.at[0]], o_vmem)

        pltpu.emit_pipeline(
            body,
            grid=(_N_TOK // win,),
            in_specs=[pl.BlockSpec((1, win), index_map=lambda i: (0, i))],
            out_specs=[pl.BlockSpec((win, _E_DIM), index_map=lambda i: (i, 0))],
            core_axis_name=("c", "s"),
            dimension_semantics=(pltpu.PARALLEL,),
        )(i_hbm, o_hbm)

    return k(W, idx2d)


def kernel(z, W):
    wt = W.T
    zsq = jnp.sum(z ** 2, axis=1, keepdims=True)
    wsq = jnp.sum(W ** 2, axis=1)[None, :]
    idx, pxp = _tc_argmin(z, wt, zsq, wsq)
    z_q = _sc_gather(W, idx.reshape(1, _N_TOK))
    return (z_q, pxp.reshape(()))


# speedup vs baseline: 1.3943x; 1.3943x over previous
import jax
import jax.numpy as jnp
from jax.experimental import pallas as pl
from jax.experimental.pallas import tpu as pltpu
from jax.experimental.pallas import tpu_sc as plsc

_N_E = 8192
_E_DIM = 32
_N_TOK = 32768

_TILE = 512
_CHUNK = 2048
_HALF = _N_E // 2


def _argmin_body(z_ref, wt_ref, zsq_ref, wsq_ref, idx_ref, pxp_ref, counts_ref):
    step = pl.program_id(0)
    nsteps = pl.num_programs(0)

    @pl.when(step == 0)
    def _():
        counts_ref[...] = jnp.zeros_like(counts_ref)

    z = z_ref[...]
    zsq = zsq_ref[...]

    halves = []
    for h in range(2):
        best_val = jnp.full((_TILE, 1), jnp.inf, dtype=jnp.float32)
        best_idx = jnp.zeros((_TILE, 1), dtype=jnp.int32)
        for cc in range(_HALF // _CHUNK):
            c = h * (_HALF // _CHUNK) + cc
            wt_c = wt_ref[:, c * _CHUNK:(c + 1) * _CHUNK]
            wsq_c = wsq_ref[:, c * _CHUNK:(c + 1) * _CHUNK]
            mm = jnp.dot(z.astype(jnp.bfloat16), wt_c.astype(jnp.bfloat16),
                         preferred_element_type=jnp.float32)
            d = zsq + wsq_c - 2.0 * mm
            m = jnp.min(d, axis=1, keepdims=True)
            iota = jax.lax.broadcasted_iota(jnp.int32, d.shape, 1) + c * _CHUNK
            i = jnp.min(jnp.where(d == m, iota, jnp.int32(_N_E)), axis=1,
                        keepdims=True)
            upd = m < best_val
            best_val = jnp.where(upd, m, best_val)
            best_idx = jnp.where(upd, i, best_idx)
        halves.append((best_val, best_idx))

    (m1, i1), (m2, i2) = halves
    b0 = m1.astype(jnp.bfloat16).astype(jnp.float32)
    win2 = m2 < b0
    best_idx = jnp.where(win2, i2, i1)

    idx_ref[...] = best_idx

    for c in range(_N_E // _CHUNK):
        iota = jax.lax.broadcasted_iota(jnp.int32, (_TILE, _CHUNK), 1) + c * _CHUNK
        hit = (best_idx == iota).astype(jnp.float32)
        counts_ref[:, c * _CHUNK:(c + 1) * _CHUNK] += jnp.sum(
            hit, axis=0, keepdims=True)

    @pl.when(step == nsteps - 1)
    def _():
        e_mean = counts_ref[...] * (1.0 / _N_TOK)
        ent = e_mean * jnp.log(e_mean + 1e-10)
        pxp_ref[...] = jnp.exp(-jnp.sum(ent)).reshape(1, 1)


def _tc_argmin(z, wt, zsq, wsq):
    n_tiles = _N_TOK // _TILE
    return pl.pallas_call(
        _argmin_body,
        grid=(n_tiles,),
        in_specs=[
            pl.BlockSpec((_TILE, _E_DIM), lambda i: (i, 0)),
            pl.BlockSpec((_E_DIM, _N_E), lambda i: (0, 0)),
            pl.BlockSpec((_TILE, 1), lambda i: (i, 0)),
            pl.BlockSpec((1, _N_E), lambda i: (0, 0)),
        ],
        out_specs=[
            pl.BlockSpec((_TILE, 1), lambda i: (i, 0)),
            pl.BlockSpec((1, 1), lambda i: (0, 0)),
        ],
        out_shape=[
            jax.ShapeDtypeStruct((_N_TOK, 1), jnp.int32),
            jax.ShapeDtypeStruct((1, 1), jnp.float32),
        ],
        scratch_shapes=[pltpu.VMEM((1, _N_E), jnp.float32)],
    )(z, wt, zsq, wsq)


_GROW = 128


def _sc_gather(w_pad, idx2d):
    win = 128
    mesh = plsc.VectorSubcoreMesh(core_axis_name="c", subcore_axis_name="s")

    @pl.kernel(
        out_type=jax.ShapeDtypeStruct((_N_TOK, _GROW), w_pad.dtype),
        mesh=mesh,
    )
    def k(w_hbm, i_hbm, o_hbm):
        def body(i_vmem, o_vmem):
            pltpu.sync_copy(w_hbm.at[i_vmem.at[0]], o_vmem)

        pltpu.emit_pipeline(
            body,
            grid=(_N_TOK // win,),
            in_specs=[pl.BlockSpec((1, win), index_map=lambda i: (0, i))],
            out_specs=[pl.BlockSpec((win, _GROW), index_map=lambda i: (i, 0))],
            core_axis_name=("c", "s"),
            dimension_semantics=(pltpu.PARALLEL,),
        )(i_hbm, o_hbm)

    return k(w_pad, idx2d)


def kernel(z, W):
    wt = W.T
    zsq = jnp.sum(z ** 2, axis=1, keepdims=True)
    wsq = jnp.sum(W ** 2, axis=1)[None, :]
    idx, pxp = _tc_argmin(z, wt, zsq, wsq)
    w_pad = jnp.pad(W, ((0, 0), (0, _GROW - _E_DIM)))
    zq = _sc_gather(w_pad, idx.reshape(1, _N_TOK))[:, :_E_DIM]
    z_q = z + (zq - z)
    return (z_q, pxp.reshape(()))


# speedup vs baseline: 1.3975x; 1.0023x over previous
import jax
import jax.numpy as jnp
from jax.experimental import pallas as pl
from jax.experimental.pallas import tpu as pltpu
from jax.experimental.pallas import tpu_sc as plsc

_N_E = 8192
_E_DIM = 32
_N_TOK = 32768

_TILE = 512
_CHUNK = 2048
_HALF = _N_E // 2


def _argmin_body(z_ref, wt_ref, zsq_ref, wsq_ref, idx_ref, pxp_ref, counts_ref):
    step = pl.program_id(0)
    nsteps = pl.num_programs(0)

    @pl.when(step == 0)
    def _():
        counts_ref[...] = jnp.zeros_like(counts_ref)

    z = z_ref[...]
    zb = z.astype(jnp.bfloat16)
    zsq = zsq_ref[...]
    iota_row = jax.lax.broadcasted_iota(jnp.int32, (1, _CHUNK), 1)

    halves = []
    for h in range(2):
        best_val = jnp.full((_TILE, 1), jnp.inf, dtype=jnp.float32)
        best_idx = jnp.zeros((_TILE, 1), dtype=jnp.int32)
        for cc in range(_HALF // _CHUNK):
            c = h * (_HALF // _CHUNK) + cc
            wt_c = wt_ref[:, c * _CHUNK:(c + 1) * _CHUNK]
            wsq_c = wsq_ref[:, c * _CHUNK:(c + 1) * _CHUNK]
            mm = jnp.dot(zb, wt_c.astype(jnp.bfloat16),
                         preferred_element_type=jnp.float32)
            d = zsq + wsq_c - 2.0 * mm
            m = jnp.min(d, axis=1, keepdims=True)
            i = jnp.min(jnp.where(d == m, iota_row + c * _CHUNK,
                                  jnp.int32(_N_E)), axis=1,
                        keepdims=True)
            upd = m < best_val
            best_val = jnp.where(upd, m, best_val)
            best_idx = jnp.where(upd, i, best_idx)
        halves.append((best_val, best_idx))

    (m1, i1), (m2, i2) = halves
    b0 = m1.astype(jnp.bfloat16).astype(jnp.float32)
    win2 = m2 < b0
    best_idx = jnp.where(win2, i2, i1)

    idx_ref[...] = best_idx

    for c in range(_N_E // _CHUNK):
        hit = (best_idx == iota_row + c * _CHUNK).astype(jnp.float32)
        counts_ref[:, c * _CHUNK:(c + 1) * _CHUNK] += jnp.sum(
            hit, axis=0, keepdims=True)

    @pl.when(step == nsteps - 1)
    def _():
        e_mean = counts_ref[...] * (1.0 / _N_TOK)
        ent = e_mean * jnp.log(e_mean + 1e-10)
        pxp_ref[...] = jnp.exp(-jnp.sum(ent)).reshape(1, 1)


def _tc_argmin(z, wt, zsq, wsq):
    n_tiles = _N_TOK // _TILE
    return pl.pallas_call(
        _argmin_body,
        grid=(n_tiles,),
        in_specs=[
            pl.BlockSpec((_TILE, _E_DIM), lambda i: (i, 0)),
            pl.BlockSpec((_E_DIM, _N_E), lambda i: (0, 0)),
            pl.BlockSpec((_TILE, 1), lambda i: (i, 0)),
            pl.BlockSpec((1, _N_E), lambda i: (0, 0)),
        ],
        out_specs=[
            pl.BlockSpec((_TILE, 1), lambda i: (i, 0)),
            pl.BlockSpec((1, 1), lambda i: (0, 0)),
        ],
        out_shape=[
            jax.ShapeDtypeStruct((_N_TOK, 1), jnp.int32),
            jax.ShapeDtypeStruct((1, 1), jnp.float32),
        ],
        scratch_shapes=[pltpu.VMEM((1, _N_E), jnp.float32)],
    )(z, wt, zsq, wsq)


_GROW = 128


def _sc_gather(w_pad, idx2d):
    win = 128
    mesh = plsc.VectorSubcoreMesh(core_axis_name="c", subcore_axis_name="s")

    @pl.kernel(
        out_type=jax.ShapeDtypeStruct((_N_TOK, _GROW), w_pad.dtype),
        mesh=mesh,
    )
    def k(w_hbm, i_hbm, o_hbm):
        def body(i_vmem, o_vmem):
            pltpu.sync_copy(w_hbm.at[i_vmem.at[0]], o_vmem)

        pltpu.emit_pipeline(
            body,
            grid=(_N_TOK // win,),
            in_specs=[pl.BlockSpec((1, win), index_map=lambda i: (0, i))],
            out_specs=[pl.BlockSpec((win, _GROW), index_map=lambda i: (i, 0))],
            core_axis_name=("c", "s"),
            dimension_semantics=(pltpu.PARALLEL,),
        )(i_hbm, o_hbm)

    return k(w_pad, idx2d)


def kernel(z, W):
    wt = W.T
    zsq = jnp.sum(z ** 2, axis=1, keepdims=True)
    wsq = jnp.sum(W ** 2, axis=1)[None, :]
    idx, pxp = _tc_argmin(z, wt, zsq, wsq)
    w_pad = jnp.pad(W, ((0, 0), (0, _GROW - _E_DIM)))
    zq = _sc_gather(w_pad, idx.reshape(1, _N_TOK))[:, :_E_DIM]
    z_q = z + (zq - z)
    return (z_q, pxp.reshape(()))


# speedup vs baseline: 1.4036x; 1.0044x over previous
import jax
import jax.numpy as jnp
from jax.experimental import pallas as pl
from jax.experimental.pallas import tpu as pltpu
from jax.experimental.pallas import tpu_sc as plsc

_N_E = 8192
_E_DIM = 32
_N_TOK = 32768

_TILE = 512
_CHUNK = 2048
_HALF = _N_E // 2


def _argmin_body(z_ref, wt_ref, zsq_ref, wsq_ref, idx_ref, pxp_ref, counts_ref):
    step = pl.program_id(0)
    nsteps = pl.num_programs(0)

    @pl.when(step == 0)
    def _():
        counts_ref[...] = jnp.zeros_like(counts_ref)

    z = z_ref[...]
    zb = z.astype(jnp.bfloat16)
    zsq = zsq_ref[...]
    iota_row = jax.lax.broadcasted_iota(jnp.int32, (1, _CHUNK), 1)

    halves = []
    for h in range(2):
        best_val = jnp.full((_TILE, 1), jnp.inf, dtype=jnp.float32)
        best_idx = jnp.zeros((_TILE, 1), dtype=jnp.int32)
        for cc in range(_HALF // _CHUNK):
            c = h * (_HALF // _CHUNK) + cc
            wt_c = wt_ref[:, c * _CHUNK:(c + 1) * _CHUNK]
            wsq_c = wsq_ref[:, c * _CHUNK:(c + 1) * _CHUNK]
            mm2 = jnp.dot(zb, wt_c.astype(jnp.bfloat16),
                          preferred_element_type=jnp.float32)
            d = (zsq + wsq_c) + mm2
            m = jnp.min(d, axis=1, keepdims=True)
            i = jnp.min(jnp.where(d == m, iota_row + c * _CHUNK,
                                  jnp.int32(_N_E)), axis=1,
                        keepdims=True)
            upd = m < best_val
            best_val = jnp.where(upd, m, best_val)
            best_idx = jnp.where(upd, i, best_idx)
        halves.append((best_val, best_idx))

    (m1, i1), (m2, i2) = halves
    b0 = m1.astype(jnp.bfloat16).astype(jnp.float32)
    win2 = m2 < b0
    best_idx = jnp.where(win2, i2, i1)

    idx_ref[...] = best_idx

    for c in range(_N_E // _CHUNK):
        hit = (best_idx == iota_row + c * _CHUNK).astype(jnp.float32)
        counts_ref[:, c * _CHUNK:(c + 1) * _CHUNK] += jnp.sum(
            hit, axis=0, keepdims=True)

    @pl.when(step == nsteps - 1)
    def _():
        e_mean = counts_ref[...] * (1.0 / _N_TOK)
        ent = e_mean * jnp.log(e_mean + 1e-10)
        pxp_ref[...] = jnp.exp(-jnp.sum(ent)).reshape(1, 1)


def _tc_argmin(z, wt, zsq, wsq):
    n_tiles = _N_TOK // _TILE
    return pl.pallas_call(
        _argmin_body,
        grid=(n_tiles,),
        in_specs=[
            pl.BlockSpec((_TILE, _E_DIM), lambda i: (i, 0)),
            pl.BlockSpec((_E_DIM, _N_E), lambda i: (0, 0)),
            pl.BlockSpec((_TILE, 1), lambda i: (i, 0)),
            pl.BlockSpec((1, _N_E), lambda i: (0, 0)),
        ],
        out_specs=[
            pl.BlockSpec((_TILE, 1), lambda i: (i, 0)),
            pl.BlockSpec((1, 1), lambda i: (0, 0)),
        ],
        out_shape=[
            jax.ShapeDtypeStruct((_N_TOK, 1), jnp.int32),
            jax.ShapeDtypeStruct((1, 1), jnp.float32),
        ],
        scratch_shapes=[pltpu.VMEM((1, _N_E), jnp.float32)],
    )(z, wt, zsq, wsq)


_GROW = 128


def _sc_gather(w_pad, idx2d):
    win = 128
    mesh = plsc.VectorSubcoreMesh(core_axis_name="c", subcore_axis_name="s")

    @pl.kernel(
        out_type=jax.ShapeDtypeStruct((_N_TOK, _GROW), w_pad.dtype),
        mesh=mesh,
    )
    def k(w_hbm, i_hbm, o_hbm):
        def body(i_vmem, o_vmem):
            pltpu.sync_copy(w_hbm.at[i_vmem.at[0]], o_vmem)

        pltpu.emit_pipeline(
            body,
            grid=(_N_TOK // win,),
            in_specs=[pl.BlockSpec((1, win), index_map=lambda i: (0, i))],
            out_specs=[pl.BlockSpec((win, _GROW), index_map=lambda i: (i, 0))],
            core_axis_name=("c", "s"),
            dimension_semantics=(pltpu.PARALLEL,),
        )(i_hbm, o_hbm)

    return k(w_pad, idx2d)


def kernel(z, W):
    wt = -2.0 * W.T
    zsq = jnp.sum(z ** 2, axis=1, keepdims=True)
    wsq = jnp.sum(W ** 2, axis=1)[None, :]
    idx, pxp = _tc_argmin(z, wt, zsq, wsq)
    w_pad = jnp.pad(W, ((0, 0), (0, _GROW - _E_DIM)))
    zq = _sc_gather(w_pad, idx.reshape(1, _N_TOK))[:, :_E_DIM]
    z_q = z + (zq - z)
    return (z_q, pxp.reshape(()))
